# baseline (speedup 1.0000x reference)
import functools

import jax
import jax.numpy as jnp
from jax import lax
from jax.experimental import pallas as pl
from jax.experimental.pallas import tpu as pltpu
from jax.experimental.pallas import tpu_sc as plsc

_N = 4096
_R = 256
_G = _N // _R

_NC = 2
_NS = 16
_NW = _NC * _NS
_PW = _N // _NW


def _sc_gather_body(preds_hbm, tgt_hbm, t_out, d_out, tidx_v, didx_v,
                    tval_v, dval_v, sem):
    wid = lax.axis_index("s") * _NC + lax.axis_index("c")
    base = wid * _PW
    pltpu.sync_copy(tgt_hbm.at[pl.ds(base, _PW)], tidx_v)
    for j in range(_PW // 16):
        lane = lax.iota(jnp.int32, 16)
        r = base + j * 16 + lane
        tv = tidx_v[pl.ds(j * 16, 16)]
        tidx_v[pl.ds(j * 16, 16)] = r * _N + tv
        didx_v[pl.ds(j * 16, 16)] = r * _N + r
    pltpu.async_copy(preds_hbm.at[tidx_v], tval_v, sem).wait()
    pltpu.async_copy(preds_hbm.at[didx_v], dval_v, sem).wait()
    pltpu.sync_copy(tval_v, t_out.at[pl.ds(base, _PW)])
    pltpu.sync_copy(dval_v, d_out.at[pl.ds(base, _PW)])


_sc_gather = functools.partial(
    pl.kernel,
    mesh=plsc.VectorSubcoreMesh(core_axis_name="c", subcore_axis_name="s"),
    out_type=(
        jax.ShapeDtypeStruct((_N,), jnp.float32),
        jax.ShapeDtypeStruct((_N,), jnp.float32),
    ),
    scratch_types=[
        pltpu.VMEM((_PW,), jnp.int32),
        pltpu.VMEM((_PW,), jnp.int32),
        pltpu.VMEM((_PW,), jnp.float32),
        pltpu.VMEM((_PW,), jnp.float32),
        pltpu.SemaphoreType.DMA,
    ],
)(_sc_gather_body)


def _tc_body(x_ref, tgt_ref, d_ref, t_ref, out_ref):
    i = pl.program_id(0)
    x = x_ref[...]
    neg = jnp.float32(-jnp.inf)
    m0 = jnp.max(x, axis=1, keepdims=True)
    m1 = jnp.max(jnp.where(x < m0, x, neg), axis=1, keepdims=True)
    m2 = jnp.max(jnp.where(x < m1, x, neg), axis=1, keepdims=True)
    s = jnp.sum(jnp.exp(x - m0), axis=1, keepdims=True)
    lse = m0 + jnp.log(s)
    rowid = i * _R + lax.broadcasted_iota(jnp.int32, (_R, 1), 0)
    tb = tgt_ref[...]
    d = d_ref[...]
    t = t_ref[...]
    is0 = d == m0
    is1 = d == m1
    va = jnp.where(is0, m1, m0)
    vb = jnp.where(is0 | is1, m2, m1)
    ind = ((tb != rowid) & (t != va) & (t != vb)).astype(jnp.float32)
    loss = lse * (1.0 + ind) - (0.7 * d + 0.2 * va + 0.1 * vb + ind * t)
    part = jnp.sum(loss, axis=0, keepdims=True) * jnp.float32(1.0 / _N)
    prev = jnp.where(i == 0, jnp.zeros_like(part), out_ref[...])
    out_ref[...] = prev + part


def kernel(preds, targets):
    tgt = targets.astype(jnp.int32)
    t, d = _sc_gather(preds.reshape(-1), tgt)
    out = pl.pallas_call(
        _tc_body,
        grid=(_G,),
        in_specs=[
            pl.BlockSpec((_R, _N), lambda i: (i, 0)),
            pl.BlockSpec((_R, 1), lambda i: (i, 0)),
            pl.BlockSpec((_R, 1), lambda i: (i, 0)),
            pl.BlockSpec((_R, 1), lambda i: (i, 0)),
        ],
        out_specs=pl.BlockSpec((1, 1), lambda i: (0, 0)),
        out_shape=jax.ShapeDtypeStruct((1, 1), jnp.float32),
    )(preds, tgt.reshape(_N, 1), d.reshape(_N, 1), t.reshape(_N, 1))
    return out[0, 0]


# speedup vs baseline: 1.6658x; 1.6658x over previous
import functools

import jax
import jax.numpy as jnp
from jax import lax
from jax.experimental import pallas as pl
from jax.experimental.pallas import tpu as pltpu
from jax.experimental.pallas import tpu_sc as plsc

_N = 4096
_R = 256
_G = _N // _R

_NC = 2
_NS = 16
_NW = _NC * _NS
_PW = _N // _NW


def _sc_diag_body(preds_hbm, d_out, blk_v, dval_v):
    wid = lax.axis_index("s") * _NC + lax.axis_index("c")
    base = wid * _PW
    lane = lax.iota(jnp.int32, 16)
    for j in range(_PW // 16):
        b0 = base + j * 16
        pltpu.sync_copy(preds_hbm.at[pl.ds(b0, 16), pl.ds(base, _PW)], blk_v)
        acc = jnp.zeros((16,), jnp.float32)
        for l in range(16):
            acc = jnp.where(lane == l, blk_v[l, pl.ds(j * 16, 16)], acc)
        dval_v[pl.ds(j * 16, 16)] = acc
    pltpu.sync_copy(dval_v, d_out.at[pl.ds(base, _PW)])


_sc_diag = functools.partial(
    pl.kernel,
    mesh=plsc.VectorSubcoreMesh(core_axis_name="c", subcore_axis_name="s"),
    out_type=jax.ShapeDtypeStruct((_N,), jnp.float32),
    scratch_types=[
        pltpu.VMEM((16, _PW), jnp.float32),
        pltpu.VMEM((_PW,), jnp.float32),
    ],
)(_sc_diag_body)


def _tc_body(x_ref, tgt_ref, d_ref, out_ref):
    i = pl.program_id(0)
    x = x_ref[...]
    neg = jnp.float32(-jnp.inf)
    m0 = jnp.max(x, axis=1, keepdims=True)
    m1 = jnp.max(jnp.where(x < m0, x, neg), axis=1, keepdims=True)
    m2 = jnp.max(jnp.where(x < m1, x, neg), axis=1, keepdims=True)
    s = jnp.sum(jnp.exp(x - m0), axis=1, keepdims=True)
    lse = m0 + jnp.log(s)
    rowid = i * _R + lax.broadcasted_iota(jnp.int32, (_R, 1), 0)
    tb = tgt_ref[...]
    d = d_ref[...]
    col = lax.broadcasted_iota(jnp.int32, (_R, _N), 1)
    t = jnp.sum(jnp.where(col == tb, x, 0.0), axis=1, keepdims=True)
    is0 = d == m0
    is1 = d == m1
    va = jnp.where(is0, m1, m0)
    vb = jnp.where(is0 | is1, m2, m1)
    ind = ((tb != rowid) & (t != va) & (t != vb)).astype(jnp.float32)
    loss = lse * (1.0 + ind) - (0.7 * d + 0.2 * va + 0.1 * vb + ind * t)
    part = jnp.sum(loss, axis=0, keepdims=True) * jnp.float32(1.0 / _N)
    prev = jnp.where(i == 0, jnp.zeros_like(part), out_ref[...])
    out_ref[...] = prev + part


def kernel(preds, targets):
    tgt = targets.astype(jnp.int32)
    d = _sc_diag(preds)
    out = pl.pallas_call(
        _tc_body,
        grid=(_G,),
        in_specs=[
            pl.BlockSpec((_R, _N), lambda i: (i, 0)),
            pl.BlockSpec((_R, 1), lambda i: (i, 0)),
            pl.BlockSpec((_R, 1), lambda i: (i, 0)),
        ],
        out_specs=pl.BlockSpec((1, 1), lambda i: (0, 0)),
        out_shape=jax.ShapeDtypeStruct((1, 1), jnp.float32),
    )(preds, tgt.reshape(_N, 1), d.reshape(_N, 1))
    return out[0, 0]
